# Initial kernel scaffold; baseline (speedup 1.0000x reference)
import functools

import jax
import jax.numpy as jnp
from jax import lax
from jax.experimental import pallas as pl
from jax.experimental.pallas import tpu as pltpu
from jax.experimental.pallas import tpu_sc as plsc

_N = 10000
_E = 320000
_NC = 2
_NS = 16
_NW = _NC * _NS
_C = 80
_EPT = _E // _NW
_NCHUNK = _EPT // _C
_NPT = _N // _NS

_mesh = plsc.VectorSubcoreMesh(core_axis_name="c", subcore_axis_name="s")

_f32 = jnp.float32


def _iota16():
    return lax.iota(jnp.int32, 16)


def _leaky_exp(x):
    return jnp.exp(jnp.maximum(x, 0.2 * x))


@functools.partial(
    pl.kernel,
    out_type=(
        jax.ShapeDtypeStruct((_E, 8), _f32),
        jax.ShapeDtypeStruct((_NC, _N, 8), _f32),
    ),
    mesh=_mesh,
    scratch_types=[
        pltpu.VMEM((_C,), jnp.int32),
        pltpu.VMEM((_C,), jnp.int32),
        pltpu.VMEM((_C, 16), _f32),
        pltpu.VMEM((_C, 16), _f32),
        pltpu.VMEM((_C, 8), _f32),
        pltpu.VMEM((_NPT, 8), _f32),
        pltpu.VMEM_SHARED((_N, 8), _f32),
        pltpu.SemaphoreType.DMA,
        pltpu.SemaphoreType.DMA,
    ],
)
def _pass_a0(src_hbm, dst_hbm, lr_hbm, zeros_hbm, ee_hbm, dp_hbm,
             srcv, dstv, lrs, lrd, eev, zbuf, dacc, sem0, sem1):
    c = lax.axis_index("c")
    s = lax.axis_index("s")
    wid = c * _NS + s
    i16 = _iota16()
    row01 = i16 // 8
    col07 = jnp.bitwise_and(i16, 7)
    col8f = col07 + 8

    pltpu.sync_copy(zeros_hbm.at[pl.ds(s * _NPT, _NPT)], zbuf)
    pltpu.sync_copy(zbuf, dacc.at[pl.ds(s * _NPT, _NPT)])
    plsc.subcore_barrier()

    def chunk(ci, carry):
        base = wid * _EPT + ci * _C
        pltpu.sync_copy(src_hbm.at[pl.ds(base, _C)], srcv)
        pltpu.sync_copy(dst_hbm.at[pl.ds(base, _C)], dstv)
        cp0 = pltpu.async_copy(lr_hbm.at[srcv], lrs, sem0)
        cp1 = pltpu.async_copy(lr_hbm.at[dstv], lrd, sem1)
        cp0.wait()
        cp1.wait()

        def pair(j, carry2):
            row = row01 + 2 * j
            el2 = plsc.load_gather(lrs, [row, col07])
            er2 = plsc.load_gather(lrd, [row, col8f])
            ee = _leaky_exp(el2 + er2)
            plsc.store_scatter(eev, [row, col07], ee)
            return carry2

        lax.fori_loop(0, _C // 2, pair, 0)
        pltpu.sync_copy(eev, ee_hbm.at[pl.ds(base, _C)])
        pltpu.sync_copy(eev, dacc.at[dstv], add=True)
        return carry

    lax.fori_loop(0, _NCHUNK, chunk, 0)
    plsc.subcore_barrier()
    pltpu.sync_copy(dacc.at[pl.ds(s * _NPT, _NPT)], zbuf)
    pltpu.sync_copy(zbuf, dp_hbm.at[c, pl.ds(s * _NPT, _NPT)])


@functools.partial(
    pl.kernel,
    out_type=jax.ShapeDtypeStruct((_NC, _N, 64), _f32),
    mesh=_mesh,
    scratch_types=[
        pltpu.VMEM((_C,), jnp.int32),
        pltpu.VMEM((_C,), jnp.int32),
        pltpu.VMEM((_C, 8), _f32),
        pltpu.VMEM((_C, 8), _f32),
        pltpu.VMEM((_C, 8), _f32),
        pltpu.VMEM((_C, 64), _f32),
        pltpu.VMEM((_C, 64), _f32),
        pltpu.VMEM((_NPT, 64), _f32),
        pltpu.VMEM_SHARED((_N, 64), _f32),
        pltpu.SemaphoreType.DMA,
        pltpu.SemaphoreType.DMA,
    ],
)
def _pass_b0(src_hbm, dst_hbm, ee_hbm, dinv_hbm, feat_hbm, zeros_hbm, out_hbm,
             srcv, dstv, eev, dinvg, acoef, frows, msgb, zbuf, oacc,
             sem0, sem1):
    c = lax.axis_index("c")
    s = lax.axis_index("s")
    wid = c * _NS + s
    i16 = _iota16()
    row01 = i16 // 8
    col07 = jnp.bitwise_and(i16, 7)

    pltpu.sync_copy(zeros_hbm.at[pl.ds(s * _NPT, _NPT)], zbuf)
    pltpu.sync_copy(zbuf, oacc.at[pl.ds(s * _NPT, _NPT)])
    plsc.subcore_barrier()

    def chunk(ci, carry):
        base = wid * _EPT + ci * _C
        pltpu.sync_copy(src_hbm.at[pl.ds(base, _C)], srcv)
        pltpu.sync_copy(dst_hbm.at[pl.ds(base, _C)], dstv)
        cp0 = pltpu.async_copy(feat_hbm.at[srcv], frows, sem0)
        cp1 = pltpu.async_copy(dinv_hbm.at[dstv], dinvg, sem1)
        pltpu.sync_copy(ee_hbm.at[pl.ds(base, _C)], eev)
        cp0.wait()
        cp1.wait()

        def pair(j, carry2):
            row = row01 + 2 * j
            a = (plsc.load_gather(eev, [row, col07])
                 * plsc.load_gather(dinvg, [row, col07]))
            plsc.store_scatter(acoef, [row, col07], a)
            return carry2

        lax.fori_loop(0, _C // 2, pair, 0)

        def edge(e, carry2):
            rowe = jnp.full((16,), e, jnp.int32)
            for m in range(4):
                colm = 16 * m + i16
                aexp = plsc.load_gather(acoef, [rowe, 2 * m + row01])
                fr = plsc.load_gather(frows, [rowe, colm])
                plsc.store_scatter(msgb, [rowe, colm], fr * aexp)
            return carry2

        lax.fori_loop(0, _C, edge, 0)
        pltpu.sync_copy(msgb, oacc.at[dstv], add=True)
        return carry

    lax.fori_loop(0, _NCHUNK, chunk, 0)
    plsc.subcore_barrier()
    pltpu.sync_copy(oacc.at[pl.ds(s * _NPT, _NPT)], zbuf)
    pltpu.sync_copy(zbuf, out_hbm.at[c, pl.ds(s * _NPT, _NPT)])


@functools.partial(
    pl.kernel,
    out_type=(
        jax.ShapeDtypeStruct((_E,), _f32),
        jax.ShapeDtypeStruct((_NC, _N, 8), _f32),
    ),
    mesh=_mesh,
    scratch_types=[
        pltpu.VMEM((_C,), jnp.int32),
        pltpu.VMEM((_C,), jnp.int32),
        pltpu.VMEM((_C, 16), _f32),
        pltpu.VMEM((_C, 16), _f32),
        pltpu.VMEM((_C,), _f32),
        pltpu.VMEM((_C, 8), _f32),
        pltpu.VMEM((_NPT, 8), _f32),
        pltpu.VMEM_SHARED((_N, 8), _f32),
        pltpu.SemaphoreType.DMA,
        pltpu.SemaphoreType.DMA,
    ],
)
def _pass_a1(src_hbm, dst_hbm, lr_hbm, zeros_hbm, ee_hbm, dp_hbm,
             srcv, dstv, lrs, lrd, eev1, eev8, zbuf, dacc, sem0, sem1):
    c = lax.axis_index("c")
    s = lax.axis_index("s")
    wid = c * _NS + s
    i16 = _iota16()
    zero16 = jnp.zeros((16,), jnp.int32)
    one16 = zero16 + 1

    pltpu.sync_copy(zeros_hbm.at[pl.ds(s * _NPT, _NPT)], zbuf)
    pltpu.sync_copy(zbuf, dacc.at[pl.ds(s * _NPT, _NPT)])
    pltpu.sync_copy(zeros_hbm.at[pl.ds(0, _C)], eev8)
    plsc.subcore_barrier()

    def chunk(ci, carry):
        base = wid * _EPT + ci * _C
        pltpu.sync_copy(src_hbm.at[pl.ds(base, _C)], srcv)
        pltpu.sync_copy(dst_hbm.at[pl.ds(base, _C)], dstv)
        cp0 = pltpu.async_copy(lr_hbm.at[srcv], lrs, sem0)
        cp1 = pltpu.async_copy(lr_hbm.at[dstv], lrd, sem1)
        cp0.wait()
        cp1.wait()

        def grp(j, carry2):
            rows = 16 * j + i16
            el = plsc.load_gather(lrs, [rows, zero16])
            er = plsc.load_gather(lrd, [rows, one16])
            ee = _leaky_exp(el + er)
            eev1[pl.ds(16 * j, 16)] = ee
            plsc.store_scatter(eev8, [rows, zero16], ee)
            return carry2

        lax.fori_loop(0, _C // 16, grp, 0)
        pltpu.sync_copy(eev1, ee_hbm.at[pl.ds(base, _C)])
        pltpu.sync_copy(eev8, dacc.at[dstv], add=True)
        return carry

    lax.fori_loop(0, _NCHUNK, chunk, 0)
    plsc.subcore_barrier()
    pltpu.sync_copy(dacc.at[pl.ds(s * _NPT, _NPT)], zbuf)
    pltpu.sync_copy(zbuf, dp_hbm.at[c, pl.ds(s * _NPT, _NPT)])


@functools.partial(
    pl.kernel,
    out_type=jax.ShapeDtypeStruct((_NC, _N, 48), _f32),
    mesh=_mesh,
    scratch_types=[
        pltpu.VMEM((_C,), jnp.int32),
        pltpu.VMEM((_C,), jnp.int32),
        pltpu.VMEM((_C,), _f32),
        pltpu.VMEM((_C, 8), _f32),
        pltpu.VMEM((_C,), _f32),
        pltpu.VMEM((_C, 48), _f32),
        pltpu.VMEM((_C, 48), _f32),
        pltpu.VMEM((_NPT, 48), _f32),
        pltpu.VMEM_SHARED((_N, 48), _f32),
        pltpu.SemaphoreType.DMA,
        pltpu.SemaphoreType.DMA,
    ],
)
def _pass_b1(src_hbm, dst_hbm, ee_hbm, dinv_hbm, feat_hbm, zeros_hbm, out_hbm,
             srcv, dstv, eev1, dinvg, acoef, frows, msgb, zbuf, oacc,
             sem0, sem1):
    c = lax.axis_index("c")
    s = lax.axis_index("s")
    wid = c * _NS + s
    i16 = _iota16()
    zero16 = jnp.zeros((16,), jnp.int32)

    pltpu.sync_copy(zeros_hbm.at[pl.ds(s * _NPT, _NPT)], zbuf)
    pltpu.sync_copy(zbuf, oacc.at[pl.ds(s * _NPT, _NPT)])
    plsc.subcore_barrier()

    def chunk(ci, carry):
        base = wid * _EPT + ci * _C
        pltpu.sync_copy(src_hbm.at[pl.ds(base, _C)], srcv)
        pltpu.sync_copy(dst_hbm.at[pl.ds(base, _C)], dstv)
        cp0 = pltpu.async_copy(feat_hbm.at[srcv], frows, sem0)
        cp1 = pltpu.async_copy(dinv_hbm.at[dstv], dinvg, sem1)
        pltpu.sync_copy(ee_hbm.at[pl.ds(base, _C)], eev1)
        cp0.wait()
        cp1.wait()

        def grp(j, carry2):
            rows = 16 * j + i16
            dinv = plsc.load_gather(dinvg, [rows, zero16])
            acoef[pl.ds(16 * j, 16)] = eev1[pl.ds(16 * j, 16)] * dinv
            return carry2

        lax.fori_loop(0, _C // 16, grp, 0)

        def edge(e, carry2):
            rowe = jnp.full((16,), e, jnp.int32)
            aexp = plsc.load_gather(acoef, [rowe])
            for m in range(3):
                colm = 16 * m + i16
                fr = plsc.load_gather(frows, [rowe, colm])
                plsc.store_scatter(msgb, [rowe, colm], fr * aexp)
            return carry2

        lax.fori_loop(0, _C, edge, 0)
        pltpu.sync_copy(msgb, oacc.at[dstv], add=True)
        return carry

    lax.fori_loop(0, _NCHUNK, chunk, 0)
    plsc.subcore_barrier()
    pltpu.sync_copy(oacc.at[pl.ds(s * _NPT, _NPT)], zbuf)
    pltpu.sync_copy(zbuf, out_hbm.at[c, pl.ds(s * _NPT, _NPT)])


_BLK = 1000
_GRID = _N // _BLK


def _tc0_body(h_ref, w0_ref, al_ref, ar_ref, feat_ref, lr_ref):
    feat = jnp.dot(h_ref[...], w0_ref[...], preferred_element_type=_f32)
    feat_ref[...] = feat
    el = jnp.dot(feat, al_ref[...], preferred_element_type=_f32)
    er = jnp.dot(feat, ar_ref[...], preferred_element_type=_f32)
    lr_ref[...] = jnp.concatenate([el, er], axis=1)


_tc0 = pl.pallas_call(
    _tc0_body,
    grid=(_GRID,),
    in_specs=[
        pl.BlockSpec((_BLK, 128), lambda i: (i, 0)),
        pl.BlockSpec((128, 64), lambda i: (0, 0)),
        pl.BlockSpec((64, 8), lambda i: (0, 0)),
        pl.BlockSpec((64, 8), lambda i: (0, 0)),
    ],
    out_specs=[
        pl.BlockSpec((_BLK, 64), lambda i: (i, 0)),
        pl.BlockSpec((_BLK, 16), lambda i: (i, 0)),
    ],
    out_shape=[
        jax.ShapeDtypeStruct((_N, 64), _f32),
        jax.ShapeDtypeStruct((_N, 16), _f32),
    ],
)


def _tc_comb_body(dp_ref, dinv_ref):
    dinv_ref[...] = 1.0 / (dp_ref[0] + dp_ref[1] + 1e-9)


_tc_comb = pl.pallas_call(
    _tc_comb_body,
    grid=(_GRID,),
    in_specs=[pl.BlockSpec((2, _BLK, 8), lambda i: (0, i, 0))],
    out_specs=pl.BlockSpec((_BLK, 8), lambda i: (i, 0)),
    out_shape=jax.ShapeDtypeStruct((_N, 8), _f32),
)


def _tc2_body(op_ref, b0_ref, w1_ref, alr_ref, feat_ref, lr_ref):
    x = op_ref[0] + op_ref[1] + b0_ref[...]
    h1 = jnp.where(x > 0, x, jnp.exp(jnp.minimum(x, 0.0)) - 1.0)
    feat = jnp.dot(h1, w1_ref[...], preferred_element_type=_f32)
    feat_ref[...] = feat
    lr_ref[...] = jnp.dot(feat, alr_ref[...], preferred_element_type=_f32)


_tc2 = pl.pallas_call(
    _tc2_body,
    grid=(_GRID,),
    in_specs=[
        pl.BlockSpec((2, _BLK, 64), lambda i: (0, i, 0)),
        pl.BlockSpec((1, 64), lambda i: (0, 0)),
        pl.BlockSpec((64, 48), lambda i: (0, 0)),
        pl.BlockSpec((48, 16), lambda i: (0, 0)),
    ],
    out_specs=[
        pl.BlockSpec((_BLK, 48), lambda i: (i, 0)),
        pl.BlockSpec((_BLK, 16), lambda i: (i, 0)),
    ],
    out_shape=[
        jax.ShapeDtypeStruct((_N, 48), _f32),
        jax.ShapeDtypeStruct((_N, 16), _f32),
    ],
)


def _tc4_body(op_ref, b1_ref, out_ref):
    out_ref[...] = op_ref[0, :, :40] + op_ref[1, :, :40] + b1_ref[...]


_tc4 = pl.pallas_call(
    _tc4_body,
    grid=(_GRID,),
    in_specs=[
        pl.BlockSpec((2, _BLK, 48), lambda i: (0, i, 0)),
        pl.BlockSpec((1, 40), lambda i: (0, 0)),
    ],
    out_specs=pl.BlockSpec((_BLK, 40), lambda i: (i, 0)),
    out_shape=jax.ShapeDtypeStruct((_N, 40), _f32),
)


def kernel(h, edge_index, W0, attn_l0, attn_r0, b0, W1, attn_l1, attn_r1, b1):
    src = edge_index[0].astype(jnp.int32)
    dst = edge_index[1].astype(jnp.int32)

    eye8 = jnp.eye(8, dtype=_f32)
    al0 = (attn_l0[:, :, None] * eye8[:, None, :]).reshape(64, 8)
    ar0 = (attn_r0[:, :, None] * eye8[:, None, :]).reshape(64, 8)
    w1p = jnp.pad(W1, ((0, 0), (0, 8)))
    alr1 = jnp.zeros((48, 16), _f32)
    alr1 = alr1.at[:40, 0].set(attn_l1[0]).at[:40, 1].set(attn_r1[0])

    z8 = jnp.zeros((_N, 8), _f32)
    z48 = jnp.zeros((_N, 48), _f32)
    z64 = jnp.zeros((_N, 64), _f32)

    feat0, lr0 = _tc0(h, W0, al0, ar0)
    ee0, dp0 = _pass_a0(src, dst, lr0, z8)
    dinv0 = _tc_comb(dp0)
    op0 = _pass_b0(src, dst, ee0, dinv0, feat0, z64)
    feat1, lr1 = _tc2(op0, b0.reshape(1, 64), w1p, alr1)
    ee1, dp1 = _pass_a1(src, dst, lr1, z8)
    dinv1 = _tc_comb(dp1)
    op1 = _pass_b1(src, dst, ee1, dinv1, feat1, z48)
    return _tc4(op1, b1.reshape(1, 40))


# speedup vs baseline: 41.8990x; 41.8990x over previous
import functools

import jax
import jax.numpy as jnp
from jax import lax
from jax.experimental import pallas as pl
from jax.experimental.pallas import tpu as pltpu
from jax.experimental.pallas import tpu_sc as plsc

_N = 10000
_E = 320000
_NC = 2
_NS = 16
_NW = _NC * _NS
_C = 80
_EPT = _E // _NW
_NCHUNK = _EPT // _C
_NP = 10240
_NPT = _NP // _NS

_mesh = plsc.VectorSubcoreMesh(core_axis_name="c", subcore_axis_name="s")
_CP = pltpu.CompilerParams(needs_layout_passes=False)

_f32 = jnp.float32


def _iota16():
    return lax.iota(jnp.int32, 16)


def _leaky_exp(x):
    return jnp.exp(jnp.maximum(x, 0.2 * x))


@functools.partial(
    pl.kernel,
    out_type=jax.ShapeDtypeStruct((_NC, _NP, 72), _f32),
    mesh=_mesh,
    compiler_params=_CP,
    scratch_types=[
        pltpu.VMEM((_C,), jnp.int32),
        pltpu.VMEM((_C,), jnp.int32),
        pltpu.VMEM((_C, 128), _f32),
        pltpu.VMEM((_C, 128), _f32),
        pltpu.VMEM((_C, 72), _f32),
        pltpu.VMEM_SHARED((_NP, 72), _f32),
        pltpu.SemaphoreType.DMA,
        pltpu.SemaphoreType.DMA,
    ],
)
def _layer0(src_hbm, dst_hbm, t0_hbm, zeros_hbm, out_hbm,
            srcv, dstv, srow, drow, msgb, acc, sem0, sem1):
    c = lax.axis_index("c")
    s = lax.axis_index("s")
    wid = c * _NS + s
    i16 = _iota16()
    row01 = i16 // 8
    col07 = jnp.bitwise_and(i16, 7)
    sl = pl.ds(s * _NPT, _NPT)

    pltpu.sync_copy(zeros_hbm.at[sl], acc.at[sl])
    plsc.subcore_barrier()

    def chunk(ci, carry):
        base = wid * _EPT + ci * _C
        pltpu.sync_copy(src_hbm.at[pl.ds(base, _C)], srcv)
        pltpu.sync_copy(dst_hbm.at[pl.ds(base, _C)], dstv)
        cps = pltpu.async_copy(t0_hbm.at[srcv], srow, sem0)
        cpd = pltpu.async_copy(t0_hbm.at[dstv], drow, sem1)
        cps.wait()
        cpd.wait()

        def pair(j, carry2):
            row = row01 + 2 * j
            el2 = plsc.load_gather(srow, [row, col07 + 64])
            er2 = plsc.load_gather(drow, [row, col07 + 72])
            ee = _leaky_exp(el2 + er2)
            plsc.store_scatter(msgb, [row, col07 + 64], ee)
            return carry2

        lax.fori_loop(0, _C // 2, pair, 0)

        def edge(e, carry2):
            rowe = jnp.full((16,), e, jnp.int32)
            for m in range(4):
                colm = 16 * m + i16
                aexp = plsc.load_gather(msgb, [rowe, 64 + 2 * m + row01])
                fr = plsc.load_gather(srow, [rowe, colm])
                plsc.store_scatter(msgb, [rowe, colm], fr * aexp)
            return carry2

        lax.fori_loop(0, _C, edge, 0)
        pltpu.sync_copy(msgb, acc.at[dstv], add=True)
        return carry

    lax.fori_loop(0, _NCHUNK, chunk, 0)
    plsc.subcore_barrier()
    pltpu.sync_copy(acc.at[sl], out_hbm.at[c, sl])


@functools.partial(
    pl.kernel,
    out_type=jax.ShapeDtypeStruct((_NC, _NP, 48), _f32),
    mesh=_mesh,
    compiler_params=_CP,
    scratch_types=[
        pltpu.VMEM((_C,), jnp.int32),
        pltpu.VMEM((_C,), jnp.int32),
        pltpu.VMEM((_C, 128), _f32),
        pltpu.VMEM((_C, 128), _f32),
        pltpu.VMEM((_C, 48), _f32),
        pltpu.VMEM_SHARED((_NP, 48), _f32),
        pltpu.SemaphoreType.DMA,
        pltpu.SemaphoreType.DMA,
    ],
)
def _layer1(src_hbm, dst_hbm, t1_hbm, zeros_hbm, out_hbm,
            srcv, dstv, srow, drow, msgb, acc, sem0, sem1):
    c = lax.axis_index("c")
    s = lax.axis_index("s")
    wid = c * _NS + s
    i16 = _iota16()
    zero16 = jnp.zeros((16,), jnp.int32)
    f40 = zero16 + 40
    f41 = zero16 + 41
    msk8 = i16 < 8
    sl = pl.ds(s * _NPT, _NPT)

    pltpu.sync_copy(zeros_hbm.at[sl], acc.at[sl])
    pltpu.sync_copy(zeros_hbm.at[pl.ds(0, _C)], msgb)
    plsc.subcore_barrier()

    def chunk(ci, carry):
        base = wid * _EPT + ci * _C
        pltpu.sync_copy(src_hbm.at[pl.ds(base, _C)], srcv)
        pltpu.sync_copy(dst_hbm.at[pl.ds(base, _C)], dstv)
        cps = pltpu.async_copy(t1_hbm.at[srcv], srow, sem0)
        cpd = pltpu.async_copy(t1_hbm.at[dstv], drow, sem1)
        cps.wait()
        cpd.wait()

        def grp(j, carry2):
            rows = 16 * j + i16
            el = plsc.load_gather(srow, [rows, f40])
            er = plsc.load_gather(drow, [rows, f41])
            ee = _leaky_exp(el + er)
            plsc.store_scatter(msgb, [rows, f40], ee)
            return carry2

        lax.fori_loop(0, _C // 16, grp, 0)

        def edge(e, carry2):
            rowe = jnp.full((16,), e, jnp.int32)
            aexp = plsc.load_gather(msgb, [rowe, f40])
            for m in range(3):
                colm = 16 * m + i16
                if m < 2:
                    fr = plsc.load_gather(srow, [rowe, colm])
                    plsc.store_scatter(msgb, [rowe, colm], fr * aexp)
                else:
                    fr = plsc.load_gather(srow, [rowe, colm], mask=msk8)
                    plsc.store_scatter(msgb, [rowe, colm], fr * aexp,
                                       mask=msk8)
            return carry2

        lax.fori_loop(0, _C, edge, 0)
        pltpu.sync_copy(msgb, acc.at[dstv], add=True)
        return carry

    lax.fori_loop(0, _NCHUNK, chunk, 0)
    plsc.subcore_barrier()
    pltpu.sync_copy(acc.at[sl], out_hbm.at[c, sl])


_BLK = 2000
_GRID = _N // _BLK


def _tc0_body(h_ref, w0_ref, al_ref, ar_ref, t0_ref):
    feat = jnp.dot(h_ref[...], w0_ref[...], preferred_element_type=_f32)
    el = jnp.dot(feat, al_ref[...], preferred_element_type=_f32)
    er = jnp.dot(feat, ar_ref[...], preferred_element_type=_f32)
    pad = jnp.zeros((_BLK, 48), _f32)
    t0_ref[...] = jnp.concatenate([feat, el, er, pad], axis=1)


_tc0 = pl.pallas_call(
    _tc0_body,
    grid=(_GRID,),
    in_specs=[
        pl.BlockSpec((_BLK, 128), lambda i: (i, 0)),
        pl.BlockSpec((128, 64), lambda i: (0, 0)),
        pl.BlockSpec((64, 8), lambda i: (0, 0)),
        pl.BlockSpec((64, 8), lambda i: (0, 0)),
    ],
    out_specs=pl.BlockSpec((_BLK, 128), lambda i: (i, 0)),
    out_shape=jax.ShapeDtypeStruct((_NP, 128), _f32),
)


def _tc2_body(op_ref, b0_ref, w1_ref, alr_ref, e864_ref, t1_ref):
    x = op_ref[0] + op_ref[1]
    dinv = 1.0 / (x[:, 64:72] + 1e-9)
    dinv64 = jnp.dot(dinv, e864_ref[...], preferred_element_type=_f32)
    z = x[:, :64] * dinv64 + b0_ref[...]
    h1 = jnp.where(z > 0, z, jnp.exp(jnp.minimum(z, 0.0)) - 1.0)
    feat = jnp.dot(h1, w1_ref[...], preferred_element_type=_f32)
    lr = jnp.dot(feat, alr_ref[...], preferred_element_type=_f32)
    pad = jnp.zeros((_BLK, 80), _f32)
    t1_ref[...] = jnp.concatenate([feat, lr, pad], axis=1)


_tc2 = pl.pallas_call(
    _tc2_body,
    grid=(_GRID,),
    in_specs=[
        pl.BlockSpec((2, _BLK, 72), lambda i: (0, i, 0)),
        pl.BlockSpec((1, 64), lambda i: (0, 0)),
        pl.BlockSpec((64, 40), lambda i: (0, 0)),
        pl.BlockSpec((40, 8), lambda i: (0, 0)),
        pl.BlockSpec((8, 64), lambda i: (0, 0)),
    ],
    out_specs=pl.BlockSpec((_BLK, 128), lambda i: (i, 0)),
    out_shape=jax.ShapeDtypeStruct((_NP, 128), _f32),
)


def _tc4_body(op_ref, b1_ref, out_ref):
    x = op_ref[0] + op_ref[1]
    out_ref[...] = x[:, :40] / (x[:, 40:41] + 1e-9) + b1_ref[...]


_tc4 = pl.pallas_call(
    _tc4_body,
    grid=(_GRID,),
    in_specs=[
        pl.BlockSpec((2, _BLK, 48), lambda i: (0, i, 0)),
        pl.BlockSpec((1, 40), lambda i: (0, 0)),
    ],
    out_specs=pl.BlockSpec((_BLK, 40), lambda i: (i, 0)),
    out_shape=jax.ShapeDtypeStruct((_N, 40), _f32),
)


def kernel(h, edge_index, W0, attn_l0, attn_r0, b0, W1, attn_l1, attn_r1, b1):
    src = edge_index[0].astype(jnp.int32)
    dst = edge_index[1].astype(jnp.int32)

    eye8 = jnp.eye(8, dtype=_f32)
    al0 = (attn_l0[:, :, None] * eye8[:, None, :]).reshape(64, 8)
    ar0 = (attn_r0[:, :, None] * eye8[:, None, :]).reshape(64, 8)
    alr1 = jnp.zeros((40, 8), _f32)
    alr1 = alr1.at[:, 0].set(attn_l1[0]).at[:, 1].set(attn_r1[0])
    e864 = (eye8[:, :, None] * jnp.ones((1, 1, 8), _f32)).reshape(8, 64)

    z72 = jnp.zeros((_NP, 72), _f32)
    z48 = jnp.zeros((_NP, 48), _f32)

    t0 = _tc0(h, W0, al0, ar0)
    op0 = _layer0(src, dst, t0, z72)
    t1 = _tc2(op0, b0.reshape(1, 64), W1, alr1, e864)
    op1 = _layer1(src, dst, t1, z48)
    return _tc4(op1, b1.reshape(1, 40))
